# baseline (device time: 18090 ns/iter reference)
import os

import jax
import jax.numpy as jnp
from jax import lax
from jax.experimental import pallas as pl
from jax.experimental.pallas import tpu as pltpu

N_DEV = 16

_ABLATE = os.environ.get("ABLATE", "full")


def kernel(x, w_mat):
    m_per, k = x.shape
    n = w_mat.shape[1]
    n_per = n // N_DEV

    def body(x_ref, w_ref, out_ref, y_ref, recv_ref, send_sems, recv_sems):
        my = lax.axis_index("i")
        ack_sem = pltpu.get_barrier_semaphore()

        y = jnp.dot(x_ref[:, :], w_ref[:, :], preferred_element_type=jnp.float32)
        y = y * jax.nn.sigmoid(y)
        for j in range(N_DEV):
            y_ref[j, :, :] = y[:, j * n_per:(j + 1) * n_per]

        if _ABLATE in ("normr", "nobar"):
            out_ref[pl.ds(my * m_per, m_per), :] = y_ref[my, :, :]
            return

        rdmas = []
        for d in range(1, N_DEV):
            tgt = lax.rem(my + d, N_DEV)
            rdma = pltpu.make_async_remote_copy(
                src_ref=y_ref.at[tgt],
                dst_ref=recv_ref.at[my],
                send_sem=send_sems.at[d],
                recv_sem=recv_sems.at[my],
                device_id=(tgt,),
                device_id_type=pl.DeviceIdType.MESH,
            )
            rdma.start()
            rdmas.append(rdma)

        out_ref[pl.ds(my * m_per, m_per), :] = y_ref[my, :, :]

        for d in range(1, N_DEV):
            src = lax.rem(my - d + N_DEV, N_DEV)
            recv = pltpu.make_async_remote_copy(
                src_ref=y_ref.at[src],
                dst_ref=recv_ref.at[src],
                send_sem=send_sems.at[d],
                recv_sem=recv_sems.at[src],
                device_id=(src,),
                device_id_type=pl.DeviceIdType.MESH,
            )
            recv.wait_recv()
            out_ref[pl.ds(src * m_per, m_per), :] = recv_ref[src, :, :]
            if _ABLATE != "noack":
                pl.semaphore_signal(
                    ack_sem, inc=1,
                    device_id=(src,), device_id_type=pl.DeviceIdType.MESH,
                )

        for rdma in rdmas:
            rdma.wait_send()
        if _ABLATE != "noack":
            pl.semaphore_wait(ack_sem, N_DEV - 1)

    return pl.pallas_call(
        body,
        out_shape=jax.ShapeDtypeStruct((N_DEV * m_per, n_per), jnp.float32),
        in_specs=[
            pl.BlockSpec(memory_space=pltpu.VMEM),
            pl.BlockSpec(memory_space=pltpu.VMEM),
        ],
        out_specs=pl.BlockSpec(memory_space=pltpu.VMEM),
        scratch_shapes=[
            pltpu.VMEM((N_DEV, m_per, n_per), jnp.float32),
            pltpu.VMEM((N_DEV, m_per, n_per), jnp.float32),
            pltpu.SemaphoreType.DMA((N_DEV,)),
            pltpu.SemaphoreType.DMA((N_DEV,)),
        ],
        compiler_params=pltpu.CompilerParams(collective_id=0),
    )(x, w_mat)


# device time: 17752 ns/iter; 1.0190x vs baseline; 1.0190x over previous
import jax
import jax.numpy as jnp
from jax import lax
from jax.experimental import pallas as pl
from jax.experimental.pallas import tpu as pltpu

N_DEV = 16
N_BLK = 4
PER_BLK = N_DEV // N_BLK
_ORDER = [2, 3, 1, 0]


def kernel(x, w_mat):
    m_per, k = x.shape
    n = w_mat.shape[1]
    n_per = n // N_DEV
    n_blk = n // N_BLK

    def body(x_ref, w_ref, out_ref, y_ref, recv_ref,
             send_sems, recv_sems, copy_sems):
        my = lax.axis_index("i")
        ack_sem = pltpu.get_barrier_semaphore()
        my_plane = lax.div(my, PER_BLK)

        rdmas = []
        for s in range(N_BLK):
            b = lax.rem(my_plane + _ORDER[s], N_BLK)
            yb = jnp.dot(
                x_ref[:, :],
                w_ref[:, pl.ds(b * n_blk, n_blk)],
                preferred_element_type=jnp.float32,
            )
            yb = yb * jax.nn.sigmoid(yb)
            for j in range(PER_BLK):
                t = PER_BLK * b + j
                y_ref[t, :, :] = yb[:, j * n_per:(j + 1) * n_per]
                rdma = pltpu.make_async_remote_copy(
                    src_ref=y_ref.at[t],
                    dst_ref=recv_ref.at[my],
                    send_sem=send_sems.at[PER_BLK * s + j],
                    recv_sem=recv_sems.at[my],
                    device_id=(t,),
                    device_id_type=pl.DeviceIdType.MESH,
                )
                rdma.start()
                rdmas.append(rdma)

        copies = []
        for d in range(N_DEV):
            src = lax.rem(my - d + N_DEV, N_DEV)
            recv = pltpu.make_async_remote_copy(
                src_ref=y_ref.at[src],
                dst_ref=recv_ref.at[src],
                send_sem=send_sems.at[0],
                recv_sem=recv_sems.at[src],
                device_id=(src,),
                device_id_type=pl.DeviceIdType.MESH,
            )
            recv.wait_recv()
            cp = pltpu.make_async_copy(
                recv_ref.at[src],
                out_ref.at[pl.ds(src * m_per, m_per), :],
                copy_sems.at[d],
            )
            cp.start()
            copies.append((cp, src))

        for cp, src in copies:
            cp.wait()
            pl.semaphore_signal(
                ack_sem, inc=1,
                device_id=(src,), device_id_type=pl.DeviceIdType.MESH,
            )

        for rdma in rdmas:
            rdma.wait_send()
        pl.semaphore_wait(ack_sem, N_DEV)

    return pl.pallas_call(
        body,
        out_shape=jax.ShapeDtypeStruct((N_DEV * m_per, n_per), jnp.float32),
        in_specs=[
            pl.BlockSpec(memory_space=pltpu.VMEM),
            pl.BlockSpec(memory_space=pltpu.VMEM),
        ],
        out_specs=pl.BlockSpec(memory_space=pltpu.VMEM),
        scratch_shapes=[
            pltpu.VMEM((N_DEV, m_per, n_per), jnp.float32),
            pltpu.VMEM((N_DEV, m_per, n_per), jnp.float32),
            pltpu.SemaphoreType.DMA((N_DEV,)),
            pltpu.SemaphoreType.DMA((N_DEV,)),
            pltpu.SemaphoreType.DMA((N_DEV,)),
        ],
        compiler_params=pltpu.CompilerParams(collective_id=0),
    )(x, w_mat)
